# Initial kernel scaffold; baseline (speedup 1.0000x reference)
import jax
import jax.numpy as jnp
from jax.experimental import pallas as pl


def kernel(phi1_indices, phi1_values, phi1_inv_indices, phi1_inv_values, phi2_indices, phi2_values, phi2_inv_indices, phi2_inv_values, fea, joblst, weight1, diag1, weight2, diag2, rnn_Wih, rnn_Whh, rnn_bih, rnn_bhh, dense_W, dense_b, item_factors):
    raise NotImplementedError("write your pallas kernel here")


# speedup vs baseline: 1.0313x; 1.0313x over previous
import functools

import jax
import jax.numpy as jnp
from jax.experimental import pallas as pl
from jax.experimental.pallas import tpu as pltpu


def _rnn_body(emb_ref, wih_ref, whh_ref, bias_ref, out_ref, hcur, hnxt):
    t = pl.program_id(0)
    j = pl.program_id(1)
    nj = pl.num_programs(1)

    @pl.when(jnp.logical_and(t == 0, j == 0))
    def _():
        hcur[...] = jnp.zeros_like(hcur)

    emb = emb_ref[0]
    pre = jax.lax.dot_general(wih_ref[...], emb, (((1,), (0,)), ((), ())),
                              preferred_element_type=jnp.float32)
    rec = jax.lax.dot_general(whh_ref[...], hcur[...], (((1,), (0,)), ((), ())),
                              preferred_element_type=jnp.float32)
    blk = rec.shape[0]
    h_new = jnp.tanh(pre + rec + bias_ref[pl.ds(j * blk, blk), :])
    hnxt[pl.ds(j * blk, blk), :] = h_new
    out_ref[0, pl.ds(j * blk, blk), :] = h_new

    @pl.when(j == nj - 1)
    def _():
        hcur[...] = hnxt[...]


def _rnn(emb_pad, wih, whh, bias, blk_j=400):
    T, IN, _ = emb_pad.shape
    n = whh.shape[0]
    nj = n // blk_j
    return pl.pallas_call(
        _rnn_body,
        grid=(T, nj),
        in_specs=[
            pl.BlockSpec((1, IN, 8), lambda t, j: (t, 0, 0)),
            pl.BlockSpec((blk_j, IN), lambda t, j: (j, 0)),
            pl.BlockSpec((blk_j, n), lambda t, j: (j, 0)),
            pl.BlockSpec((n, 1), lambda t, j: (0, 0)),
        ],
        out_specs=pl.BlockSpec((1, n, 8), lambda t, j: (t, 0, 0)),
        out_shape=jax.ShapeDtypeStruct((T, n, 8), jnp.float32),
        scratch_shapes=[pltpu.VMEM((n, 8), jnp.float32),
                        pltpu.VMEM((n, 8), jnp.float32)],
    )(emb_pad, wih, whh, bias)


def _mm2_body(x_ref, w1_ref, w2_ref, o1_ref, o2_ref):
    x = x_ref[...]
    o1_ref[...] = jnp.dot(x, w1_ref[...], preferred_element_type=jnp.float32)
    o2_ref[...] = jnp.dot(x, w2_ref[...], preferred_element_type=jnp.float32)


def _mm2(x, w1, w2, blk_m=2000):
    m, k = x.shape
    out = w1.shape[1]
    grid = (m // blk_m,)
    return pl.pallas_call(
        _mm2_body,
        grid=grid,
        in_specs=[
            pl.BlockSpec((blk_m, k), lambda i: (i, 0)),
            pl.BlockSpec((k, out), lambda i: (0, 0)),
            pl.BlockSpec((k, out), lambda i: (0, 0)),
        ],
        out_specs=[pl.BlockSpec((blk_m, out), lambda i: (i, 0)),
                   pl.BlockSpec((blk_m, out), lambda i: (i, 0))],
        out_shape=[jax.ShapeDtypeStruct((m, out), jnp.float32),
                   jax.ShapeDtypeStruct((m, out), jnp.float32)],
    )(x, w1, w2)


def _gelu_exact(x):
    return x * 0.5 * (1.0 + jax.lax.erf(x * (2.0 ** -0.5)))


def _fin_body(res_ref, emb_ref, w_ref, b_ref, out_ref):
    x = res_ref[0]
    d = jax.lax.dot_general(x, w_ref[...], (((1,), (1,)), ((), ())),
                            preferred_element_type=jnp.float32) + b_ref[...]
    g = _gelu_exact(d)
    e = _gelu_exact(emb_ref[0])
    out_ref[0] = jax.nn.sigmoid(g * e)


def _fin(res, emb_col, w, b, blk_n=2000):
    bsz, n, f = res.shape
    k = w.shape[0]
    return pl.pallas_call(
        _fin_body,
        grid=(bsz, n // blk_n),
        in_specs=[
            pl.BlockSpec((1, blk_n, f), lambda b_, i: (b_, i, 0)),
            pl.BlockSpec((1, blk_n, 1), lambda b_, i: (b_, i, 0)),
            pl.BlockSpec((k, f), lambda b_, i: (0, 0)),
            pl.BlockSpec((1, k), lambda b_, i: (0, 0)),
        ],
        out_specs=pl.BlockSpec((1, blk_n, k), lambda b_, i: (b_, i, 0)),
        out_shape=jax.ShapeDtypeStruct((bsz, n, k), jnp.float32),
    )(res, emb_col, w, b)


def _spmm(idx, vals, n, x):
    rows = idx[0]
    cols = idx[1]
    gathered = vals[:, None] * x[cols]
    return jax.ops.segment_sum(gathered, rows, num_segments=n)


def kernel(phi1_indices, phi1_values, phi1_inv_indices, phi1_inv_values,
           phi2_indices, phi2_values, phi2_inv_indices, phi2_inv_values,
           fea, joblst, weight1, diag1, weight2, diag2,
           rnn_Wih, rnn_Whh, rnn_bih, rnn_bhh, dense_W, dense_b,
           item_factors):
    bsz, n, fin = fea.shape
    length = joblst.shape[1]

    emb_seq = jnp.transpose(item_factors[joblst], (1, 2, 0))
    emb_pad = jnp.zeros((length, emb_seq.shape[1], 8), jnp.float32)
    emb_pad = emb_pad.at[:, :, :bsz].set(emb_seq)
    bias = (rnn_bih + rnn_bhh)[:, None]
    emb_all = _rnn(emb_pad, rnn_Wih, rnn_Whh, bias)
    emb = emb_all[-1, :, :bsz].T

    f1, f2 = _mm2(fea.reshape(bsz * n, fin), weight1, weight2)
    f1 = f1.reshape(bsz, n, -1)
    f2 = f2.reshape(bsz, n, -1)

    outs = []
    for i in range(bsz):
        r1 = phi1_values[i] * diag1[phi1_indices[i][1]]
        t1 = _spmm(phi1_inv_indices[i], phi1_inv_values[i], n, f1[i])
        o1 = _spmm(phi1_indices[i], r1, n, t1)
        r2 = phi2_values[i] * diag2[phi2_indices[i][1]]
        t2 = _spmm(phi2_inv_indices[i], phi2_inv_values[i], n, f2[i])
        o2 = _spmm(phi2_indices[i], r2, n, t2)
        outs.append(o1 + o2)
    res = jnp.stack(outs, axis=0)

    return _fin(res, emb[:, :, None], dense_W, dense_b[None, :])


# speedup vs baseline: 5.6626x; 5.4908x over previous
import functools

import jax
import jax.numpy as jnp
from jax import lax
from jax.experimental import pallas as pl
from jax.experimental.pallas import tpu as pltpu
from jax.experimental.pallas import tpu_sc as plsc


def _rnn_body(emb_ref, wih_ref, whh_ref, bias_ref, out_ref, hcur, hnxt):
    t = pl.program_id(0)
    j = pl.program_id(1)
    nj = pl.num_programs(1)

    @pl.when(jnp.logical_and(t == 0, j == 0))
    def _():
        hcur[...] = jnp.zeros_like(hcur)

    emb = emb_ref[0]
    pre = jax.lax.dot_general(wih_ref[...], emb, (((1,), (0,)), ((), ())),
                              preferred_element_type=jnp.float32)
    rec = jax.lax.dot_general(whh_ref[...], hcur[...], (((1,), (0,)), ((), ())),
                              preferred_element_type=jnp.float32)
    blk = rec.shape[0]
    h_new = jnp.tanh(pre + rec + bias_ref[pl.ds(j * blk, blk), :])
    hnxt[pl.ds(j * blk, blk), :] = h_new
    out_ref[0, pl.ds(j * blk, blk), :] = h_new

    @pl.when(j == nj - 1)
    def _():
        hcur[...] = hnxt[...]


def _rnn(emb_pad, wih, whh, bias, blk_j=400):
    T, IN, _ = emb_pad.shape
    n = whh.shape[0]
    nj = n // blk_j
    return pl.pallas_call(
        _rnn_body,
        grid=(T, nj),
        in_specs=[
            pl.BlockSpec((1, IN, 8), lambda t, j: (t, 0, 0)),
            pl.BlockSpec((blk_j, IN), lambda t, j: (j, 0)),
            pl.BlockSpec((blk_j, n), lambda t, j: (j, 0)),
            pl.BlockSpec((n, 1), lambda t, j: (0, 0)),
        ],
        out_specs=pl.BlockSpec((1, n, 8), lambda t, j: (t, 0, 0)),
        out_shape=jax.ShapeDtypeStruct((T, n, 8), jnp.float32),
        scratch_shapes=[pltpu.VMEM((n, 8), jnp.float32),
                        pltpu.VMEM((n, 8), jnp.float32)],
    )(emb_pad, wih, whh, bias)


def _mm2_body(x_ref, w1a_ref, w1b_ref, w2a_ref, w2b_ref, o1_ref, o2_ref):
    x = x_ref[...]
    o1_ref[0] = jnp.dot(x, w1a_ref[...], preferred_element_type=jnp.float32)
    o1_ref[1] = jnp.dot(x, w1b_ref[...], preferred_element_type=jnp.float32)
    o2_ref[0] = jnp.dot(x, w2a_ref[...], preferred_element_type=jnp.float32)
    o2_ref[1] = jnp.dot(x, w2b_ref[...], preferred_element_type=jnp.float32)


def _mm2(x, w1a, w1b, w2a, w2b, blk_m=2000):
    m, k = x.shape
    h = w1a.shape[1]
    grid = (m // blk_m,)
    return pl.pallas_call(
        _mm2_body,
        grid=grid,
        in_specs=[
            pl.BlockSpec((blk_m, k), lambda i: (i, 0)),
            pl.BlockSpec((k, h), lambda i: (0, 0)),
            pl.BlockSpec((k, h), lambda i: (0, 0)),
            pl.BlockSpec((k, h), lambda i: (0, 0)),
            pl.BlockSpec((k, h), lambda i: (0, 0)),
        ],
        out_specs=[pl.BlockSpec((2, blk_m, h), lambda i: (0, i, 0)),
                   pl.BlockSpec((2, blk_m, h), lambda i: (0, i, 0))],
        out_shape=[jax.ShapeDtypeStruct((2, m, h), jnp.float32),
                   jax.ShapeDtypeStruct((2, m, h), jnp.float32)],
    )(x, w1a, w1b, w2a, w2b)


_SC_E = 80


_SC_C = 5


def _sc_spmm_body(i1, v1, i1i, v1i, i2, v2, i2i, v2i, f1s, f2s, d1, d2,
                  out, a_sh, t_sh, r_sh, dg_sh, gath, rows, cols, vals, dvals):
    c = lax.axis_index("c")
    s = lax.axis_index("s")
    bsz = f1s.shape[1]
    n = a_sh.shape[0]
    npr = 640
    rem = n - 15 * npr
    nsc = i1.shape[3]

    def rows_op(fn):
        @pl.when(s < 15)
        def _():
            fn(s * npr, npr)

        @pl.when(s == 15)
        def _():
            fn(15 * npr, rem)

    def zero_gath():
        def zg(i, carry):
            for q in range(4):
                gath[i, pl.ds(q * 16, 16)] = jnp.zeros((16,), jnp.float32)
            return carry
        lax.fori_loop(0, _SC_E, zg, None)

    def zero_rows(dst):
        def fn(base, m):
            for i in range(m // _SC_E):
                pltpu.sync_copy(gath, dst.at[pl.ds(base + i * _SC_E, _SC_E)])
        rows_op(fn)

    def edge_pass(ix, vx, b, src, dst, use_diag):
        def sup(sk, _):
            pltpu.sync_copy(ix.at[b, 0, s, sk], rows.at[0])
            pltpu.sync_copy(ix.at[b, 1, s, sk], cols.at[0])
            pltpu.sync_copy(vx.at[b, s, sk], vals)

            def chunk(k, carry):
                pltpu.sync_copy(src.at[cols.at[0, k]], gath)
                if use_diag:
                    pltpu.sync_copy(dg_sh.at[cols.at[0, k]], dvals)
                    for i in range(_SC_E // 16):
                        off = k * _SC_E + i * 16
                        vals[pl.ds(off, 16)] = (
                            vals[pl.ds(off, 16)] * dvals[pl.ds(i * 16, 16)])

                def sc_(e, cc):
                    zeros16 = jnp.zeros((16,), jnp.int32)
                    v16 = plsc.load_gather(vals, [zeros16 + (k * _SC_E + e)])
                    for q in range(4):
                        gath[e, pl.ds(q * 16, 16)] = (
                            gath[e, pl.ds(q * 16, 16)] * v16)
                    return cc
                lax.fori_loop(0, _SC_E, sc_, None)
                pltpu.sync_copy(gath, dst.at[rows.at[0, k]], add=True)
                return carry
            lax.fori_loop(0, _SC_C, chunk, None)
            return _
        lax.fori_loop(0, nsc, sup, None)

    chains = ((i1i, v1i, i1, v1, f1s, d1), (i2i, v2i, i2, v2, f2s, d2))
    for b in range(bsz):
        zero_gath()
        zero_rows(r_sh)
        for (ii, vi, ip, vp, fs, dgv) in chains:
            zero_gath()
            zero_rows(t_sh)
            rows_op(lambda base, m: pltpu.sync_copy(
                dgv.at[pl.ds(base, m)], dg_sh.at[pl.ds(base, m)]))
            rows_op(lambda base, m: pltpu.sync_copy(
                fs.at[c, b, pl.ds(base, m)], a_sh.at[pl.ds(base, m)]))
            plsc.subcore_barrier()
            edge_pass(ii, vi, b, a_sh, t_sh, False)
            plsc.subcore_barrier()
            edge_pass(ip, vp, b, t_sh, r_sh, True)
            plsc.subcore_barrier()
        rows_op(lambda base, m: pltpu.sync_copy(
            r_sh.at[pl.ds(base, m)], out.at[c, b, pl.ds(base, m)]))
        plsc.subcore_barrier()


def _sc_spmm(i1, v1, i1i, v1i, i2, v2, i2i, v2i, f1s, f2s, d1, d2):
    bsz, _, nnz = i1.shape
    n = d1.shape[0]
    h = f1s.shape[3]
    ept = nnz // 16
    nsc = ept // (_SC_C * _SC_E)
    rs = (bsz, 2, 16, nsc, _SC_C, _SC_E)
    vs = (bsz, 16, nsc, _SC_C * _SC_E)
    mesh = plsc.VectorSubcoreMesh(core_axis_name="c", subcore_axis_name="s")
    kern = pl.kernel(
        _sc_spmm_body,
        out_type=jax.ShapeDtypeStruct((2, bsz, n, h), jnp.float32),
        mesh=mesh,
        compiler_params=pltpu.CompilerParams(needs_layout_passes=False,
                                             use_tc_tiling_on_sc=False),
        scratch_types=[
            pltpu.VMEM_SHARED((n, h), jnp.float32),
            pltpu.VMEM_SHARED((n, h), jnp.float32),
            pltpu.VMEM_SHARED((n, h), jnp.float32),
            pltpu.VMEM_SHARED((n,), jnp.float32),
            pltpu.VMEM((_SC_E, h), jnp.float32),
            pltpu.VMEM((1, _SC_C, _SC_E), jnp.int32),
            pltpu.VMEM((1, _SC_C, _SC_E), jnp.int32),
            pltpu.VMEM((_SC_C * _SC_E,), jnp.float32),
            pltpu.VMEM((_SC_E,), jnp.float32),
        ],
    )
    return kern(i1.reshape(rs), v1.reshape(vs), i1i.reshape(rs),
                v1i.reshape(vs), i2.reshape(rs), v2.reshape(vs),
                i2i.reshape(rs), v2i.reshape(vs), f1s, f2s, d1, d2)


def _gelu_exact(x):
    return x * 0.5 * (1.0 + jax.lax.erf(x * (2.0 ** -0.5)))


def _fin_body(ra_ref, rb_ref, emb_ref, wa_ref, wb_ref, b_ref, out_ref):
    da = jax.lax.dot_general(ra_ref[0, 0], wa_ref[...], (((1,), (1,)), ((), ())),
                             preferred_element_type=jnp.float32)
    db = jax.lax.dot_general(rb_ref[0, 0], wb_ref[...], (((1,), (1,)), ((), ())),
                             preferred_element_type=jnp.float32)
    g = _gelu_exact(da + db + b_ref[...])
    e = _gelu_exact(emb_ref[0])
    out_ref[0] = jax.nn.sigmoid(g * e)


def _fin(res_s, emb_col, wa, wb, b, blk_n=2000):
    _, bsz, n, h = res_s.shape
    k = wa.shape[0]
    return pl.pallas_call(
        _fin_body,
        grid=(bsz, n // blk_n),
        in_specs=[
            pl.BlockSpec((1, 1, blk_n, h), lambda b_, i: (0, b_, i, 0)),
            pl.BlockSpec((1, 1, blk_n, h), lambda b_, i: (1, b_, i, 0)),
            pl.BlockSpec((1, blk_n, 1), lambda b_, i: (b_, i, 0)),
            pl.BlockSpec((k, h), lambda b_, i: (0, 0)),
            pl.BlockSpec((k, h), lambda b_, i: (0, 0)),
            pl.BlockSpec((1, k), lambda b_, i: (0, 0)),
        ],
        out_specs=pl.BlockSpec((1, blk_n, k), lambda b_, i: (b_, i, 0)),
        out_shape=jax.ShapeDtypeStruct((bsz, n, k), jnp.float32),
    )(res_s, res_s, emb_col, wa, wb, b)


def kernel(phi1_indices, phi1_values, phi1_inv_indices, phi1_inv_values,
           phi2_indices, phi2_values, phi2_inv_indices, phi2_inv_values,
           fea, joblst, weight1, diag1, weight2, diag2,
           rnn_Wih, rnn_Whh, rnn_bih, rnn_bhh, dense_W, dense_b,
           item_factors):
    bsz, n, fin = fea.shape
    length = joblst.shape[1]
    h = fin // 2

    emb_seq = jnp.transpose(item_factors[joblst], (1, 2, 0))
    emb_pad = jnp.zeros((length, emb_seq.shape[1], 8), jnp.float32)
    emb_pad = emb_pad.at[:, :, :bsz].set(emb_seq)
    bias = (rnn_bih + rnn_bhh)[:, None]
    emb_all = _rnn(emb_pad, rnn_Wih, rnn_Whh, bias)
    emb = emb_all[-1, :, :bsz].T

    f1s, f2s = _mm2(fea.reshape(bsz * n, fin),
                    weight1[:, :h], weight1[:, h:],
                    weight2[:, :h], weight2[:, h:])
    f1s = f1s.reshape(2, bsz, n, h)
    f2s = f2s.reshape(2, bsz, n, h)

    res_s = _sc_spmm(phi1_indices, phi1_values,
                     phi1_inv_indices, phi1_inv_values,
                     phi2_indices, phi2_values,
                     phi2_inv_indices, phi2_inv_values,
                     f1s, f2s, diag1, diag2)

    return _fin(res_s, emb[:, :, None],
                dense_W[:, :h], dense_W[:, h:], dense_b[None, :])
